# baseline (speedup 1.0000x reference)
import functools

import jax
import jax.numpy as jnp
from jax.experimental import pallas as pl
from jax.experimental.pallas import tpu as pltpu

D_MODEL = 4096
N_EXPERTS = 64
TOP_K = 8
NEG_INF = float("-inf")


def _router_body(x_ref, wt_ref, b_ref, w_out_ref, idx_out_ref):
    x = x_ref[...]
    logits = jnp.dot(x, wt_ref[...], preferred_element_type=jnp.float32)
    logits = logits + b_ref[...]

    bt = logits.shape[0]
    expert_iota = jax.lax.broadcasted_iota(jnp.int32, (bt, N_EXPERTS), 1)

    work = logits
    top_vals = []
    top_idx = []
    for _ in range(TOP_K):
        m = jnp.max(work, axis=-1, keepdims=True)
        is_max = work == m
        idx = jnp.min(jnp.where(is_max, expert_iota, N_EXPERTS), axis=-1,
                      keepdims=True)
        top_vals.append(m)
        top_idx.append(idx)
        work = jnp.where(expert_iota == idx, NEG_INF, work)

    vals = jnp.concatenate(top_vals, axis=-1)
    idxs = jnp.concatenate(top_idx, axis=-1)

    e = jnp.exp(vals - vals[:, 0:1])
    w = e / jnp.sum(e, axis=-1, keepdims=True)

    w_out_ref[...] = w
    idx_out_ref[...] = idxs


@functools.partial(jax.jit, static_argnames=("block_t",))
def _router(x2d, wt, b2d, block_t):
    t = x2d.shape[0]
    grid = (t // block_t,)
    return pl.pallas_call(
        _router_body,
        grid=grid,
        in_specs=[
            pl.BlockSpec((block_t, D_MODEL), lambda i: (i, 0)),
            pl.BlockSpec((D_MODEL, N_EXPERTS), lambda i: (0, 0)),
            pl.BlockSpec((1, N_EXPERTS), lambda i: (0, 0)),
        ],
        out_specs=[
            pl.BlockSpec((block_t, TOP_K), lambda i: (i, 0)),
            pl.BlockSpec((block_t, TOP_K), lambda i: (i, 0)),
        ],
        out_shape=[
            jax.ShapeDtypeStruct((t, TOP_K), jnp.float32),
            jax.ShapeDtypeStruct((t, TOP_K), jnp.int32),
        ],
        compiler_params=pltpu.CompilerParams(
            dimension_semantics=("arbitrary",),
        ),
    )(x2d, wt, b2d)


def kernel(da_input_thoughts, W_router, b_router):
    batch, seq, d = da_input_thoughts.shape
    x2d = da_input_thoughts.reshape(batch * seq, d)
    wt = W_router.T
    b2d = b_router.reshape(1, N_EXPERTS)
    w, idx = _router(x2d, wt, b2d, 256)
    return (w.reshape(batch, seq, TOP_K), idx.reshape(batch, seq, TOP_K))


# speedup vs baseline: 1.2681x; 1.2681x over previous
import functools

import jax
import jax.numpy as jnp
from jax.experimental import pallas as pl
from jax.experimental.pallas import tpu as pltpu

D_MODEL = 4096
N_EXPERTS = 64
TOP_K = 8
NEG_INF = float("-inf")


INT32_MIN = jnp.iinfo(jnp.int32).min


def _router_body(x_ref, wt_ref, b_ref, w_out_ref, idx_out_ref):
    x = x_ref[...]
    logits = jnp.dot(x, wt_ref[...], preferred_element_type=jnp.float32)
    logits = logits + b_ref[...]

    bt = logits.shape[0]
    expert_iota = jax.lax.broadcasted_iota(jnp.int32, (bt, N_EXPERTS), 1)

    u = jax.lax.bitcast_convert_type(logits, jnp.int32)
    key = jnp.where(u >= 0, u, u ^ jnp.int32(0x7FFFFFFF))
    key = (key & jnp.int32(~63)) | (jnp.int32(63) - expert_iota)

    work = key
    top_keys = []
    for _ in range(TOP_K):
        m = jnp.max(work, axis=-1, keepdims=True)
        top_keys.append(m)
        work = jnp.where(work == m, INT32_MIN, work)

    k8 = jnp.concatenate(top_keys, axis=-1)
    idxs = jnp.int32(63) - (k8 & jnp.int32(63))
    ub = jnp.where(k8 >= 0, k8, k8 ^ jnp.int32(0x7FFFFFFF))
    vals = jax.lax.bitcast_convert_type(ub, jnp.float32)

    e = jnp.exp(vals - vals[:, 0:1])
    w = e / jnp.sum(e, axis=-1, keepdims=True)

    w_out_ref[...] = w
    idx_out_ref[...] = idxs


@functools.partial(jax.jit, static_argnames=("block_t",))
def _router(x2d, wt, b2d, block_t):
    t = x2d.shape[0]
    grid = (t // block_t,)
    return pl.pallas_call(
        _router_body,
        grid=grid,
        in_specs=[
            pl.BlockSpec((block_t, D_MODEL), lambda i: (i, 0)),
            pl.BlockSpec((D_MODEL, N_EXPERTS), lambda i: (0, 0)),
            pl.BlockSpec((1, N_EXPERTS), lambda i: (0, 0)),
        ],
        out_specs=[
            pl.BlockSpec((block_t, TOP_K), lambda i: (i, 0)),
            pl.BlockSpec((block_t, TOP_K), lambda i: (i, 0)),
        ],
        out_shape=[
            jax.ShapeDtypeStruct((t, TOP_K), jnp.float32),
            jax.ShapeDtypeStruct((t, TOP_K), jnp.int32),
        ],
        compiler_params=pltpu.CompilerParams(
            dimension_semantics=("arbitrary",),
        ),
    )(x2d, wt, b2d)


def kernel(da_input_thoughts, W_router, b_router):
    batch, seq, d = da_input_thoughts.shape
    x2d = da_input_thoughts.reshape(batch * seq, d)
    wt = W_router.T
    b2d = b_router.reshape(1, N_EXPERTS)
    w, idx = _router(x2d, wt, b2d, 256)
    return (w.reshape(batch, seq, TOP_K), idx.reshape(batch, seq, TOP_K))
